# baseline (speedup 1.0000x reference)
import jax
import jax.numpy as jnp
from jax import lax
from jax.experimental import pallas as pl
from jax.experimental.pallas import tpu as pltpu
from jax.experimental.pallas import tpu_sc as plsc

V = 1000
VP = 1024
B = 16384
NC = 2
NS = 16
L = 16
NW = NC * NS
BPW = B // NW
CH = 16
NCH = BPW // CH
G = 128


def _lse_body(t_ref, lse_ref):
    t = t_ref[...]
    m = jnp.max(t, axis=1, keepdims=True)
    lse_ref[...] = m + jnp.log(jnp.sum(jnp.exp(t - m), axis=1, keepdims=True))


def _loss_body(p_ref, loss_ref):
    loss_ref[...] = jnp.sum(p_ref[...]).reshape(1, 1) * (1.0 / B)


CBS = 512


def _conv_body(in_ref, out_ref):
    for tc in range(7):
        out_ref[:, tc * 128:(tc + 1) * 128] = in_ref[tc, :, :]
    out_ref[:, 896:V] = in_ref[7, :, 0:V - 896]


def _sc_body(tseg_h, tflat_h, x_h, tgt_h, lse_h, out_h, part_h,
             x_v, t_v, fidx_v, gidx_v, tl_v, ls_v, rows0_v, rows1_v, acc_v,
             semg0, semg1, semw0, semw1, sem2):
    c = lax.axis_index("c")
    s = lax.axis_index("s")
    wid = s * NC + c
    base = wid * BPW
    pltpu.sync_copy(x_h.at[pl.ds(base, BPW)], x_v)
    pltpu.sync_copy(tgt_h.at[pl.ds(base, BPW)], t_v)
    for i in range(BPW // L):
        sl = pl.ds(i * L, L)
        fidx_v[sl] = x_v[sl] * V + t_v[sl]
    for ch in range(NCH):
        xv8 = x_v[pl.ds(ch * CH, L)] * 8
        for k in range(8):
            gidx_v[pl.ds(ch * 128 + k * L, L)] = xv8 + k
    hs = []
    for g in range(BPW // G):
        sl = pl.ds(g * G, G)
        hs.append(
            pltpu.async_copy(tflat_h.at[fidx_v.at[sl]], tl_v.at[sl], sem2))
        hs.append(
            pltpu.async_copy(lse_h.at[x_v.at[sl]], ls_v.at[sl], sem2))

    bufs = (rows0_v, rows1_v)
    semg = (semg0, semg1)
    semw = (semw0, semw1)

    def gather(ch):
        b = ch % 2
        return pltpu.async_copy(
            tseg_h.at[gidx_v.at[pl.ds(ch * 128, 128)]], bufs[b], semg[b])

    hg = [None, None]
    hw = [[], []]
    hg[0] = gather(0)
    for ch in range(NCH):
        b = ch % 2
        hg[b].wait()
        if ch + 1 < NCH:
            b2 = (ch + 1) % 2
            for h in hw[b2]:
                h.wait()
            hw[b2] = []
            hg[b2] = gather(ch + 1)
        for tc in range(8):
            hw[b].append(pltpu.async_copy(
                bufs[b].at[pl.ds(tc * CH, CH)],
                out_h.at[pl.ds(tc * B + base + ch * CH, CH)], semw[b]))
    for h in hs:
        h.wait()
    acc = jnp.zeros((L,), jnp.float32)
    for i in range(BPW // L):
        sl = pl.ds(i * L, L)
        acc = acc + ls_v[sl] - tl_v[sl]
    acc_v[...] = acc
    for hl in hw:
        for h in hl:
            h.wait()
    pltpu.sync_copy(acc_v, part_h.at[pl.ds(wid * L, L)])


def kernel(x, targets, embed_table):
    x = x.astype(jnp.int32)
    targets = targets.astype(jnp.int32)
    table = embed_table.astype(jnp.float32)

    lse = pl.pallas_call(
        _lse_body,
        out_shape=jax.ShapeDtypeStruct((V, 1), jnp.float32),
    )(table)

    tseg = jnp.pad(table, ((0, 0), (0, VP - V))).reshape(V * 8, 128)
    tflat = table.reshape(V * V)

    sc_call = pl.kernel(
        _sc_body,
        mesh=plsc.VectorSubcoreMesh(core_axis_name="c", subcore_axis_name="s"),
        out_type=[
            jax.ShapeDtypeStruct((8 * B, 128), jnp.float32),
            jax.ShapeDtypeStruct((NW * L,), jnp.float32),
        ],
        scratch_types=[
            pltpu.VMEM((BPW,), jnp.int32),
            pltpu.VMEM((BPW,), jnp.int32),
            pltpu.VMEM((BPW,), jnp.int32),
            pltpu.VMEM((NCH * 128,), jnp.int32),
            pltpu.VMEM((BPW,), jnp.float32),
            pltpu.VMEM((BPW,), jnp.float32),
            pltpu.VMEM((8 * CH, 128), jnp.float32),
            pltpu.VMEM((8 * CH, 128), jnp.float32),
            pltpu.VMEM((L,), jnp.float32),
            pltpu.SemaphoreType.DMA,
            pltpu.SemaphoreType.DMA,
            pltpu.SemaphoreType.DMA,
            pltpu.SemaphoreType.DMA,
            pltpu.SemaphoreType.DMA,
        ],
    )
    out2, partials = sc_call(tseg, tflat, x, targets, lse.reshape(V))

    logits = pl.pallas_call(
        _conv_body,
        grid=(B // CBS,),
        in_specs=[pl.BlockSpec((8, CBS, 128), lambda i: (0, i, 0))],
        out_specs=pl.BlockSpec((CBS, V), lambda i: (i, 0)),
        out_shape=jax.ShapeDtypeStruct((B, V), jnp.float32),
    )(out2.reshape(8, B, 128))

    loss = pl.pallas_call(
        _loss_body,
        out_shape=jax.ShapeDtypeStruct((1, 1), jnp.float32),
    )(partials.reshape(NW, L))
    return logits, loss.reshape(())


# speedup vs baseline: 1.0713x; 1.0713x over previous
import jax
import jax.numpy as jnp
from jax import lax
from jax.experimental import pallas as pl
from jax.experimental.pallas import tpu as pltpu
from jax.experimental.pallas import tpu_sc as plsc

V = 1000
VP = 1024
B = 16384
NC = 2
NS = 16
L = 16
NW = NC * NS
BPW = B // NW
CH = 32
NCH = BPW // CH
G = 128


def _lse_body(t_ref, lse_ref):
    t = t_ref[...]
    m = jnp.max(t, axis=1, keepdims=True)
    lse_ref[...] = m + jnp.log(jnp.sum(jnp.exp(t - m), axis=1, keepdims=True))


def _loss_body(p_ref, loss_ref):
    loss_ref[...] = jnp.sum(p_ref[...]).reshape(1, 1) * (1.0 / B)


CBS = 512


def _conv_body(in_ref, out_ref):
    for tc in range(7):
        out_ref[:, tc * 128:(tc + 1) * 128] = in_ref[tc, :, :]
    out_ref[:, 896:V] = in_ref[7, :, 0:V - 896]


def _sc_body(tseg_h, tflat_h, x_h, tgt_h, lse_h, out_h, part_h,
             x_v, t_v, fidx_v, tl_v, ls_v, rows0_v, rows1_v, acc_v,
             semg0, semg1, semw0, semw1, sem2):
    c = lax.axis_index("c")
    s = lax.axis_index("s")
    wid = s * NC + c
    base = wid * BPW
    pltpu.sync_copy(x_h.at[pl.ds(base, BPW)], x_v)
    pltpu.sync_copy(tgt_h.at[pl.ds(base, BPW)], t_v)
    for i in range(BPW // L):
        sl = pl.ds(i * L, L)
        fidx_v[sl] = x_v[sl] * V + t_v[sl]
    hs = []
    for g in range(BPW // G):
        sl = pl.ds(g * G, G)
        hs.append(
            pltpu.async_copy(tflat_h.at[fidx_v.at[sl]], tl_v.at[sl], sem2))
        hs.append(
            pltpu.async_copy(lse_h.at[x_v.at[sl]], ls_v.at[sl], sem2))

    bufs = (rows0_v, rows1_v)
    semg = (semg0, semg1)
    semw = (semw0, semw1)

    def gather(ch):
        b = ch % 2
        return pltpu.async_copy(
            tseg_h.at[x_v.at[pl.ds(ch * CH, CH)]], bufs[b], semg[b])

    hg = [None, None]
    hw = [[], []]
    hg[0] = gather(0)
    for ch in range(NCH):
        b = ch % 2
        hg[b].wait()
        if ch + 1 < NCH:
            b2 = (ch + 1) % 2
            for h in hw[b2]:
                h.wait()
            hw[b2] = []
            hg[b2] = gather(ch + 1)
        for tc in range(8):
            hw[b].append(pltpu.async_copy(
                bufs[b].at[:, tc],
                out_h.at[tc, pl.ds(base + ch * CH, CH)], semw[b]))
    for h in hs:
        h.wait()
    acc = jnp.zeros((L,), jnp.float32)
    for i in range(BPW // L):
        sl = pl.ds(i * L, L)
        acc = acc + ls_v[sl] - tl_v[sl]
    acc_v[...] = acc
    for hl in hw:
        for h in hl:
            h.wait()
    pltpu.sync_copy(acc_v, part_h.at[pl.ds(wid * L, L)])


def kernel(x, targets, embed_table):
    x = x.astype(jnp.int32)
    targets = targets.astype(jnp.int32)
    table = embed_table.astype(jnp.float32)

    lse = pl.pallas_call(
        _lse_body,
        out_shape=jax.ShapeDtypeStruct((V, 1), jnp.float32),
    )(table)

    tseg = jnp.pad(table, ((0, 0), (0, VP - V))).reshape(V, 8, 128)
    tflat = table.reshape(V * V)

    sc_call = pl.kernel(
        _sc_body,
        mesh=plsc.VectorSubcoreMesh(core_axis_name="c", subcore_axis_name="s"),
        out_type=[
            jax.ShapeDtypeStruct((8, B, 128), jnp.float32),
            jax.ShapeDtypeStruct((NW * L,), jnp.float32),
        ],
        scratch_types=[
            pltpu.VMEM((BPW,), jnp.int32),
            pltpu.VMEM((BPW,), jnp.int32),
            pltpu.VMEM((BPW,), jnp.int32),
            pltpu.VMEM((BPW,), jnp.float32),
            pltpu.VMEM((BPW,), jnp.float32),
            pltpu.VMEM((CH, 8, 128), jnp.float32),
            pltpu.VMEM((CH, 8, 128), jnp.float32),
            pltpu.VMEM((L,), jnp.float32),
            pltpu.SemaphoreType.DMA,
            pltpu.SemaphoreType.DMA,
            pltpu.SemaphoreType.DMA,
            pltpu.SemaphoreType.DMA,
            pltpu.SemaphoreType.DMA,
        ],
    )
    out2, partials = sc_call(tseg, tflat, x, targets, lse.reshape(V))

    logits = pl.pallas_call(
        _conv_body,
        grid=(B // CBS,),
        in_specs=[pl.BlockSpec((8, CBS, 128), lambda i: (0, i, 0))],
        out_specs=pl.BlockSpec((CBS, V), lambda i: (i, 0)),
        out_shape=jax.ShapeDtypeStruct((B, V), jnp.float32),
    )(out2)

    loss = pl.pallas_call(
        _loss_body,
        out_shape=jax.ShapeDtypeStruct((1, 1), jnp.float32),
    )(partials.reshape(NW, L))
    return logits, loss.reshape(())


# speedup vs baseline: 1.4634x; 1.3661x over previous
import jax
import jax.numpy as jnp
from jax import lax
from jax.experimental import pallas as pl
from jax.experimental.pallas import tpu as pltpu
from jax.experimental.pallas import tpu_sc as plsc

V = 1000
VP = 1024
B = 16384
NC = 2
NS = 16
L = 16
NW = NC * NS
BPW = B // NW
CH = 32
NCH = BPW // CH
G = 128


def _lse_body(t_ref, lse_ref):
    t = t_ref[...]
    m = jnp.max(t, axis=1, keepdims=True)
    lse_ref[...] = m + jnp.log(jnp.sum(jnp.exp(t - m), axis=1, keepdims=True))


def _loss_body(p_ref, loss_ref):
    loss_ref[...] = jnp.sum(p_ref[...]).reshape(1, 1) * (1.0 / B)


CBS = 512


def _conv_body(in_ref, out_ref):
    for tc in range(7):
        out_ref[tc * 128:(tc + 1) * 128, :] = in_ref[tc, :, :].T
    out_ref[896:V, :] = in_ref[7, :, 0:V - 896].T


def _sc_body(tseg_h, tflat_h, x_h, tgt_h, lse_h, out_h, part_h,
             x_v, t_v, fidx_v, tl_v, ls_v, rows0_v, rows1_v, acc_v,
             semg0, semg1, semw0, semw1, sem2):
    c = lax.axis_index("c")
    s = lax.axis_index("s")
    wid = s * NC + c
    base = wid * BPW
    pltpu.sync_copy(x_h.at[pl.ds(base, BPW)], x_v)
    pltpu.sync_copy(tgt_h.at[pl.ds(base, BPW)], t_v)
    for i in range(BPW // L):
        sl = pl.ds(i * L, L)
        fidx_v[sl] = x_v[sl] * V + t_v[sl]
    hs = []
    for g in range(BPW // G):
        sl = pl.ds(g * G, G)
        hs.append(
            pltpu.async_copy(tflat_h.at[fidx_v.at[sl]], tl_v.at[sl], sem2))
        hs.append(
            pltpu.async_copy(lse_h.at[x_v.at[sl]], ls_v.at[sl], sem2))

    bufs = (rows0_v, rows1_v)
    semg = (semg0, semg1)
    semw = (semw0, semw1)

    def gather(ch):
        b = ch % 2
        return pltpu.async_copy(
            tseg_h.at[x_v.at[pl.ds(ch * CH, CH)]], bufs[b], semg[b])

    hg = [None, None]
    hw = [[], []]
    hg[0] = gather(0)
    for ch in range(NCH):
        b = ch % 2
        hg[b].wait()
        if ch + 1 < NCH:
            b2 = (ch + 1) % 2
            for h in hw[b2]:
                h.wait()
            hw[b2] = []
            hg[b2] = gather(ch + 1)
        for tc in range(8):
            hw[b].append(pltpu.async_copy(
                bufs[b].at[:, tc],
                out_h.at[tc, pl.ds(base + ch * CH, CH)], semw[b]))
    for h in hs:
        h.wait()
    acc = jnp.zeros((L,), jnp.float32)
    for i in range(BPW // L):
        sl = pl.ds(i * L, L)
        acc = acc + ls_v[sl] - tl_v[sl]
    acc_v[...] = acc
    for hl in hw:
        for h in hl:
            h.wait()
    pltpu.sync_copy(acc_v, part_h.at[pl.ds(wid * L, L)])


def kernel(x, targets, embed_table):
    x = x.astype(jnp.int32)
    targets = targets.astype(jnp.int32)
    table = embed_table.astype(jnp.float32)

    lse = pl.pallas_call(
        _lse_body,
        out_shape=jax.ShapeDtypeStruct((V, 1), jnp.float32),
    )(table)

    tseg = jnp.pad(table, ((0, 0), (0, VP - V))).reshape(V, 8, 128)
    tflat = table.reshape(V * V)

    sc_call = pl.kernel(
        _sc_body,
        mesh=plsc.VectorSubcoreMesh(core_axis_name="c", subcore_axis_name="s"),
        out_type=[
            jax.ShapeDtypeStruct((8, B, 128), jnp.float32),
            jax.ShapeDtypeStruct((NW * L,), jnp.float32),
        ],
        scratch_types=[
            pltpu.VMEM((BPW,), jnp.int32),
            pltpu.VMEM((BPW,), jnp.int32),
            pltpu.VMEM((BPW,), jnp.int32),
            pltpu.VMEM((BPW,), jnp.float32),
            pltpu.VMEM((BPW,), jnp.float32),
            pltpu.VMEM((CH, 8, 128), jnp.float32),
            pltpu.VMEM((CH, 8, 128), jnp.float32),
            pltpu.VMEM((L,), jnp.float32),
            pltpu.SemaphoreType.DMA,
            pltpu.SemaphoreType.DMA,
            pltpu.SemaphoreType.DMA,
            pltpu.SemaphoreType.DMA,
            pltpu.SemaphoreType.DMA,
        ],
    )
    out2, partials = sc_call(tseg, tflat, x, targets, lse.reshape(V))

    logits_t = pl.pallas_call(
        _conv_body,
        grid=(B // CBS,),
        in_specs=[pl.BlockSpec((8, CBS, 128), lambda i: (0, i, 0))],
        out_specs=pl.BlockSpec((V, CBS), lambda i: (0, i)),
        out_shape=jax.ShapeDtypeStruct((V, B), jnp.float32),
    )(out2)
    logits = logits_t.T

    loss = pl.pallas_call(
        _loss_body,
        out_shape=jax.ShapeDtypeStruct((1, 1), jnp.float32),
    )(partials.reshape(NW, L))
    return logits, loss.reshape(())


# speedup vs baseline: 1.5321x; 1.0469x over previous
import jax
import jax.numpy as jnp
from jax import lax
from jax.experimental import pallas as pl
from jax.experimental.pallas import tpu as pltpu
from jax.experimental.pallas import tpu_sc as plsc

V = 1000
VP = 1024
B = 16384
NC = 2
NS = 16
L = 16
NW = NC * NS
BPW = B // NW
CH = 32
NCH = BPW // CH
G = 128


def _lse_body(t_ref, lse_ref):
    t = t_ref[...]
    m = jnp.max(t, axis=1, keepdims=True)
    lse_ref[...] = m + jnp.log(jnp.sum(jnp.exp(t - m), axis=1, keepdims=True))


def _loss_body(p_ref, loss_ref):
    loss_ref[...] = jnp.sum(p_ref[...]).reshape(1, 1) * (1.0 / B)


CBS = 1024


def _conv_body(in_ref, out_ref):
    for tc in range(7):
        out_ref[tc * 128:(tc + 1) * 128, :] = in_ref[tc, :, :].T
    out_ref[896:V, :] = in_ref[7, :, 0:V - 896].T


def _sc_body(tseg_h, tflat_h, x_h, tgt_h, lse_h, out_h, part_h,
             x_v, t_v, fidx_v, tl_v, ls_v, rows0_v, rows1_v, acc_v,
             semg0, semg1, semw0, semw1, sem2):
    c = lax.axis_index("c")
    s = lax.axis_index("s")
    wid = s * NC + c
    base = wid * BPW
    pltpu.sync_copy(x_h.at[pl.ds(base, BPW)], x_v)
    pltpu.sync_copy(tgt_h.at[pl.ds(base, BPW)], t_v)
    for i in range(BPW // L):
        sl = pl.ds(i * L, L)
        fidx_v[sl] = x_v[sl] * V + t_v[sl]
    hs = []
    for g in range(BPW // G):
        sl = pl.ds(g * G, G)
        hs.append(
            pltpu.async_copy(tflat_h.at[fidx_v.at[sl]], tl_v.at[sl], sem2))
        hs.append(
            pltpu.async_copy(lse_h.at[x_v.at[sl]], ls_v.at[sl], sem2))

    bufs = (rows0_v, rows1_v)
    semg = (semg0, semg1)
    semw = (semw0, semw1)

    def gather(ch):
        b = ch % 2
        return pltpu.async_copy(
            tseg_h.at[x_v.at[pl.ds(ch * CH, CH)]], bufs[b], semg[b])

    hg = [None, None]
    hw = [[], []]
    hg[0] = gather(0)
    for ch in range(NCH):
        b = ch % 2
        hg[b].wait()
        if ch + 1 < NCH:
            b2 = (ch + 1) % 2
            for h in hw[b2]:
                h.wait()
            hw[b2] = []
            hg[b2] = gather(ch + 1)
        for tc in range(8):
            hw[b].append(pltpu.async_copy(
                bufs[b].at[:, tc],
                out_h.at[tc, pl.ds(base + ch * CH, CH)], semw[b]))
    for h in hs:
        h.wait()
    acc = jnp.zeros((L,), jnp.float32)
    for i in range(BPW // L):
        sl = pl.ds(i * L, L)
        acc = acc + ls_v[sl] - tl_v[sl]
    acc_v[...] = acc
    for hl in hw:
        for h in hl:
            h.wait()
    pltpu.sync_copy(acc_v, part_h.at[pl.ds(wid * L, L)])


def kernel(x, targets, embed_table):
    x = x.astype(jnp.int32)
    targets = targets.astype(jnp.int32)
    table = embed_table.astype(jnp.float32)

    lse = pl.pallas_call(
        _lse_body,
        out_shape=jax.ShapeDtypeStruct((V, 1), jnp.float32),
    )(table)

    tseg = jnp.pad(table, ((0, 0), (0, VP - V))).reshape(V, 8, 128)
    tflat = table.reshape(V * V)

    sc_call = pl.kernel(
        _sc_body,
        mesh=plsc.VectorSubcoreMesh(core_axis_name="c", subcore_axis_name="s"),
        out_type=[
            jax.ShapeDtypeStruct((8, B, 128), jnp.float32),
            jax.ShapeDtypeStruct((NW * L,), jnp.float32),
        ],
        scratch_types=[
            pltpu.VMEM((BPW,), jnp.int32),
            pltpu.VMEM((BPW,), jnp.int32),
            pltpu.VMEM((BPW,), jnp.int32),
            pltpu.VMEM((BPW,), jnp.float32),
            pltpu.VMEM((BPW,), jnp.float32),
            pltpu.VMEM((CH, 8, 128), jnp.float32),
            pltpu.VMEM((CH, 8, 128), jnp.float32),
            pltpu.VMEM((L,), jnp.float32),
            pltpu.SemaphoreType.DMA,
            pltpu.SemaphoreType.DMA,
            pltpu.SemaphoreType.DMA,
            pltpu.SemaphoreType.DMA,
            pltpu.SemaphoreType.DMA,
        ],
    )
    out2, partials = sc_call(tseg, tflat, x, targets, lse.reshape(V))

    logits_t = pl.pallas_call(
        _conv_body,
        grid=(B // CBS,),
        in_specs=[pl.BlockSpec((8, CBS, 128), lambda i: (0, i, 0))],
        out_specs=pl.BlockSpec((V, CBS), lambda i: (0, i)),
        out_shape=jax.ShapeDtypeStruct((V, B), jnp.float32),
    )(out2)
    logits = logits_t.T

    loss = pl.pallas_call(
        _loss_body,
        out_shape=jax.ShapeDtypeStruct((1, 1), jnp.float32),
    )(partials.reshape(NW, L))
    return logits, loss.reshape(())
